# baseline (speedup 1.0000x reference)
import functools

import jax
import jax.numpy as jnp
import numpy as np
from jax import lax
from jax.experimental import pallas as pl

N_ROWS = 16384
N_CLASSES_K = 1000
COLD = np.float32(0.1 / 1000.0)
HOT = np.float32(np.float32(1.0 - 0.1) + COLD)

BR = 512


def _body(tgt_ref, out_ref):
    tgt = tgt_ref[0, 0, :].reshape(BR, 1)
    col = lax.broadcasted_iota(jnp.int32, (BR, N_CLASSES_K), 1)
    out_ref[...] = jnp.where(col == tgt, HOT, COLD)


def kernel(target):
    nb = N_ROWS // BR
    tgt3 = target.astype(jnp.int32).reshape(nb, 1, BR)
    out = pl.pallas_call(
        _body,
        grid=(nb,),
        in_specs=[pl.BlockSpec((1, 1, BR), lambda i: (i, 0, 0))],
        out_specs=pl.BlockSpec((BR, N_CLASSES_K), lambda i: (i, 0)),
        out_shape=jax.ShapeDtypeStruct((N_ROWS, N_CLASSES_K), jnp.float32),
    )(tgt3)
    return out
